# Initial kernel scaffold; baseline (speedup 1.0000x reference)
import jax
import jax.numpy as jnp
from jax.experimental import pallas as pl


def kernel(gs, hs, ys, params):
    raise NotImplementedError("write your pallas kernel here")


# speedup vs baseline: 1.5787x; 1.5787x over previous
import jax
import jax.numpy as jnp
from jax.experimental import pallas as pl


def _body(g_ref, h_ref, y_ref, wi_ref, bi_ref, wo_ref, bo_ref, out_ref):
    g = g_ref[0]
    h0 = h_ref[0]
    inv_rs = 1.0 / jnp.sum(g, axis=1, keepdims=True)
    u0 = jnp.dot(h0, wi_ref[...], preferred_element_type=jnp.float32)
    t0 = jnp.dot(g, u0, preferred_element_type=jnp.float32) * inv_rs + bi_ref[...]
    h1 = jnp.where(t0 > 0, t0, jnp.exp(jnp.minimum(t0, 0.0)) - 1.0)
    u1 = jnp.dot(h1, wo_ref[...], preferred_element_type=jnp.float32)
    t1 = jnp.dot(g, u1, preferred_element_type=jnp.float32) * inv_rs + bo_ref[...]
    h2 = jnp.maximum(t1, 0.0)
    d = h2 - y_ref[0]
    out_ref[...] = jnp.broadcast_to(jnp.sum(d * d), (1, 1, 128))


def kernel(gs, hs, ys, params):
    B, N, _ = gs.shape
    IN_DIM = hs.shape[-1]
    OUT_DIM = ys.shape[-1]
    Wi = params['Wi']
    Wo = params['Wo']
    L = Wi.shape[1]
    bi = params['bi'].reshape(1, L)
    bo = params['bo'].reshape(1, OUT_DIM)

    sums = pl.pallas_call(
        _body,
        grid=(B,),
        in_specs=[
            pl.BlockSpec((1, N, N), lambda b: (b, 0, 0)),
            pl.BlockSpec((1, N, IN_DIM), lambda b: (b, 0, 0)),
            pl.BlockSpec((1, N, OUT_DIM), lambda b: (b, 0, 0)),
            pl.BlockSpec((IN_DIM, L), lambda b: (0, 0)),
            pl.BlockSpec((1, L), lambda b: (0, 0)),
            pl.BlockSpec((L, OUT_DIM), lambda b: (0, 0)),
            pl.BlockSpec((1, OUT_DIM), lambda b: (0, 0)),
        ],
        out_specs=pl.BlockSpec((1, 1, 128), lambda b: (b, 0, 0)),
        out_shape=jax.ShapeDtypeStruct((B, 1, 128), jnp.float32),
    )(gs, hs, ys, Wi, bi, Wo, bo)

    return jnp.sum(sums[:, 0, 0]) / (B * N * OUT_DIM)


# speedup vs baseline: 1.5847x; 1.0039x over previous
import jax
import jax.numpy as jnp
from jax.experimental import pallas as pl


def _body(g_ref, h_ref, y_ref, wi_ref, bi_ref, wo_ref, bo_ref, out_ref):
    g = g_ref[0]
    h0 = h_ref[0]
    inv_rs = 1.0 / jnp.sum(g, axis=1, keepdims=True)
    gb = g.astype(jnp.bfloat16)
    u0 = jnp.dot(h0, wi_ref[...], preferred_element_type=jnp.float32)
    t0 = jnp.dot(gb, u0.astype(jnp.bfloat16),
                 preferred_element_type=jnp.float32) * inv_rs + bi_ref[...]
    h1 = jnp.where(t0 > 0, t0, jnp.exp(jnp.minimum(t0, 0.0)) - 1.0)
    u1 = jnp.dot(h1, wo_ref[...], preferred_element_type=jnp.float32)
    t1 = jnp.dot(gb, u1.astype(jnp.bfloat16),
                 preferred_element_type=jnp.float32) * inv_rs + bo_ref[...]
    h2 = jnp.maximum(t1, 0.0)
    d = h2 - y_ref[0]
    out_ref[...] = jnp.broadcast_to(jnp.sum(d * d), (1, 1, 128))


def kernel(gs, hs, ys, params):
    B, N, _ = gs.shape
    IN_DIM = hs.shape[-1]
    OUT_DIM = ys.shape[-1]
    Wi = params['Wi']
    Wo = params['Wo']
    L = Wi.shape[1]
    bi = params['bi'].reshape(1, L)
    bo = params['bo'].reshape(1, OUT_DIM)

    sums = pl.pallas_call(
        _body,
        grid=(B,),
        in_specs=[
            pl.BlockSpec((1, N, N), lambda b: (b, 0, 0)),
            pl.BlockSpec((1, N, IN_DIM), lambda b: (b, 0, 0)),
            pl.BlockSpec((1, N, OUT_DIM), lambda b: (b, 0, 0)),
            pl.BlockSpec((IN_DIM, L), lambda b: (0, 0)),
            pl.BlockSpec((1, L), lambda b: (0, 0)),
            pl.BlockSpec((L, OUT_DIM), lambda b: (0, 0)),
            pl.BlockSpec((1, OUT_DIM), lambda b: (0, 0)),
        ],
        out_specs=pl.BlockSpec((1, 1, 128), lambda b: (b, 0, 0)),
        out_shape=jax.ShapeDtypeStruct((B, 1, 128), jnp.float32),
    )(gs, hs, ys, Wi, bi, Wo, bo)

    return jnp.sum(sums[:, 0, 0]) / (B * N * OUT_DIM)
